# baseline (speedup 1.0000x reference)
import functools
import jax
import jax.numpy as jnp
from jax import lax
from jax.experimental import pallas as pl
from jax.experimental.pallas import tpu as pltpu
from jax.experimental.pallas import tpu_sc as plsc

N = 10000
E = 320000
D = 128
R = 8

NC = 2
NS = 16
NW = NC * NS

B = 128
EPW = 10240
NB = EPW // B
E_PAD = NW * EPW
CH = 8
NCHUNK = NB // CH

S_PAD = 81920
SEG_DUMP = N * R
CPS = S_PAD // NS

AGG_N = 10240
APS = AGG_N // NS

_mesh = plsc.VectorSubcoreMesh(core_axis_name="c", subcore_axis_name="s")


def _fill1d(ref, n, val):
  def body(i, carry):
    ref[pl.ds(i * 16, 16)] = jnp.full((16,), val, dtype=ref.dtype)
    return carry
  lax.fori_loop(0, n // 16, body, 0)


def _zero_rows(ref):
  def body(i, carry):
    e = i // (D // 16)
    k = i % (D // 16)
    ref[e, pl.ds(k * 16, 16)] = jnp.zeros((16,), dtype=ref.dtype)
    return carry
  lax.fori_loop(0, B * (D // 16), body, 0)


@functools.partial(
    pl.kernel,
    out_type=jax.ShapeDtypeStruct((NC, S_PAD), jnp.float32),
    mesh=_mesh,
    scratch_types=[
        pltpu.VMEM((B,), jnp.int32),
        pltpu.VMEM((B,), jnp.float32),
        pltpu.VMEM((B,), jnp.float32),
        pltpu.VMEM_SHARED((S_PAD,), jnp.float32),
    ],
    compiler_params=pltpu.CompilerParams(needs_layout_passes=False),
)
def _sc_counts(seg_hbm, cnt_hbm, seg_v, ones_v, zero_v, cnt_sh):
  c = lax.axis_index("c")
  s = lax.axis_index("s")
  w = c * NS + s

  _fill1d(zero_v, B, 0.0)
  _fill1d(ones_v, B, 1.0)

  def zero_body(k, carry):
    pltpu.sync_copy(zero_v, cnt_sh.at[pl.ds(s * CPS + k * B, B)])
    return carry
  lax.fori_loop(0, CPS // B, zero_body, 0)
  plsc.subcore_barrier()

  def acc_body(b, carry):
    pltpu.sync_copy(seg_hbm.at[pl.ds(w * EPW + b * B, B)], seg_v)
    pltpu.sync_copy(ones_v, cnt_sh.at[seg_v], add=True)
    return carry
  lax.fori_loop(0, NB, acc_body, 0)
  plsc.subcore_barrier()

  pltpu.sync_copy(cnt_sh.at[pl.ds(s * CPS, CPS)], cnt_hbm.at[c, pl.ds(s * CPS, CPS)])


@functools.partial(
    pl.kernel,
    out_type=jax.ShapeDtypeStruct((NC, AGG_N, D), jnp.float32),
    mesh=_mesh,
    scratch_types=[
        pltpu.VMEM((CH, 3, B), jnp.int32),
        pltpu.VMEM((2, B), jnp.float32),
        pltpu.VMEM((2, B, D), jnp.float32),
        pltpu.SemaphoreType.DMA,
        pltpu.SemaphoreType.DMA,
        pltpu.SemaphoreType.DMA,
        pltpu.SemaphoreType.DMA,
        pltpu.SemaphoreType.DMA,
        pltpu.SemaphoreType.DMA,
        pltpu.VMEM_SHARED((AGG_N, D), jnp.float32),
    ],
    compiler_params=pltpu.CompilerParams(needs_layout_passes=False),
)
def _sc_scatter(xw_hbm, idx_hbm, inv_hbm, agg_hbm,
                idx_v, sc_v, rows_v, g0, g1, s0, s1, w0, w1, agg_sh):
  c = lax.axis_index("c")
  s = lax.axis_index("s")
  w = c * NS + s
  gsem = (g0, g1)
  ssem = (s0, s1)
  wsem = (w0, w1)

  _zero_rows(rows_v.at[0])
  def zero_body(k, carry):
    pltpu.sync_copy(rows_v.at[0], agg_sh.at[pl.ds(s * APS + k * B, B)])
    return carry
  lax.fori_loop(0, APS // B, zero_body, 0)

  plsc.subcore_barrier()

  def scale_rows(p):
    def one_edge(e, carry2):
      spl = plsc.load_gather(sc_v.at[p], [jnp.full((16,), e, jnp.int32)])
      for k in range(D // 16):
        rows_v[p, e, pl.ds(k * 16, 16)] = rows_v[p, e, pl.ds(k * 16, 16)] * spl
      return carry2
    lax.fori_loop(0, B, one_edge, 0, unroll=4)

  def chunk_body(ci, carry):
    pltpu.sync_copy(idx_hbm.at[pl.ds((w * NCHUNK + ci) * CH, CH)], idx_v)
    for pr in range(CH // 2):
      cps = []
      for p in range(2):
        b = 2 * pr + p
        rcp = pltpu.async_copy(xw_hbm.at[idx_v.at[b, 0]], rows_v.at[p],
                               gsem[p])
        scp = pltpu.async_copy(inv_hbm.at[idx_v.at[b, 1]], sc_v.at[p],
                               ssem[p])
        cps.append((rcp, scp))
      wcps = []
      for p in range(2):
        b = 2 * pr + p
        rcp, scp = cps[p]
        scp.wait()
        rcp.wait()
        scale_rows(p)
        wcps.append(pltpu.async_copy(
            rows_v.at[p], agg_sh.at[idx_v.at[b, 2]], wsem[p], add=True))
      for p in range(2):
        wcps[p].wait()
    return carry
  lax.fori_loop(0, NCHUNK, chunk_body, 0)
  plsc.subcore_barrier()

  def dump_body(k, carry):
    pltpu.sync_copy(agg_sh.at[pl.ds(s * APS + k * B, B)],
                    agg_hbm.at[c, pl.ds(s * APS + k * B, B)])
    return carry
  lax.fori_loop(0, APS // B, dump_body, 0)


def _tc_xw_body(x_ref, w_ref, out_ref):
  out_ref[0] = jnp.dot(x_ref[...], w_ref[0],
                       preferred_element_type=jnp.float32)


def _tc_xw(x, W):
  return pl.pallas_call(
      _tc_xw_body,
      grid=(R,),
      in_specs=[
          pl.BlockSpec((N, D), lambda r: (0, 0)),
          pl.BlockSpec((1, D, D), lambda r: (r, 0, 0)),
      ],
      out_specs=pl.BlockSpec((1, N, D), lambda r: (r, 0, 0)),
      out_shape=jax.ShapeDtypeStruct((R, N, D), jnp.float32),
  )(x, W)


def _tc_inv_body(cnt_ref, inv_ref):
  c = cnt_ref[0] + cnt_ref[1]
  inv_ref[...] = 1.0 / jnp.maximum(c, 1.0)


def _tc_inv(counts):
  return pl.pallas_call(
      _tc_inv_body,
      out_shape=jax.ShapeDtypeStruct((S_PAD // 128, 128), jnp.float32),
  )(counts.reshape(2, S_PAD // 128, 128)).reshape(S_PAD)


def _tc_combine_body(agg_ref, x_ref, root_ref, b_ref, out_ref, *, relu):
  acc = agg_ref[0, :N, :] + agg_ref[1, :N, :]
  acc = acc + jnp.dot(x_ref[...], root_ref[...],
                      preferred_element_type=jnp.float32) + b_ref[...]
  out_ref[...] = jnp.maximum(acc, 0.0) if relu else acc


def _tc_combine(agg, x, root, b, relu):
  return pl.pallas_call(
      functools.partial(_tc_combine_body, relu=relu),
      out_shape=jax.ShapeDtypeStruct((N, D), jnp.float32),
  )(agg, x, root, b.reshape(1, D))


def kernel(x, edge_index, edge_type, W1, root1, b1, W2, root2, b2):
  src = edge_index[0].astype(jnp.int32)
  dst = edge_index[1].astype(jnp.int32)
  et = edge_type.astype(jnp.int32)

  pad = E_PAD - E
  widx_p = jnp.concatenate([et * N + src, jnp.zeros((pad,), jnp.int32)])
  seg_p = jnp.concatenate([dst * R + et, jnp.full((pad,), SEG_DUMP, jnp.int32)])
  dst_p = jnp.concatenate([dst, jnp.full((pad,), N, jnp.int32)])
  idx_pack = jnp.stack([widx_p.reshape(-1, B), seg_p.reshape(-1, B),
                        dst_p.reshape(-1, B)], axis=1)

  counts = _sc_counts(seg_p)
  inv = _tc_inv(counts)

  xw1 = _tc_xw(x, W1).reshape(R * N, D)
  agg1 = _sc_scatter(xw1, idx_pack, inv)
  h = _tc_combine(agg1, x, root1, b1, relu=True)

  xw2 = _tc_xw(h, W2).reshape(R * N, D)
  agg2 = _sc_scatter(xw2, idx_pack, inv)
  out = _tc_combine(agg2, h, root2, b2, relu=False)
  return out


# speedup vs baseline: 1.0414x; 1.0414x over previous
import functools
import jax
import jax.numpy as jnp
from jax import lax
from jax.experimental import pallas as pl
from jax.experimental.pallas import tpu as pltpu
from jax.experimental.pallas import tpu_sc as plsc

N = 10000
E = 320000
D = 128
R = 8

NC = 2
NS = 16
NW = NC * NS

B = 128
EPW = 10240
NB = EPW // B
E_PAD = NW * EPW
CH = 8
NCHUNK = NB // CH

S_PAD = 81920
SEG_DUMP = N * R
CPS = S_PAD // NS

AGG_N = 10240
APS = AGG_N // NS

_mesh = plsc.VectorSubcoreMesh(core_axis_name="c", subcore_axis_name="s")


def _fill1d(ref, n, val):
  def body(i, carry):
    ref[pl.ds(i * 16, 16)] = jnp.full((16,), val, dtype=ref.dtype)
    return carry
  lax.fori_loop(0, n // 16, body, 0)


def _zero_rows(ref):
  def body(i, carry):
    e = i // (D // 16)
    k = i % (D // 16)
    ref[e, pl.ds(k * 16, 16)] = jnp.zeros((16,), dtype=ref.dtype)
    return carry
  lax.fori_loop(0, B * (D // 16), body, 0)


@functools.partial(
    pl.kernel,
    out_type=jax.ShapeDtypeStruct((NC, S_PAD), jnp.float32),
    mesh=_mesh,
    scratch_types=[
        pltpu.VMEM((B,), jnp.int32),
        pltpu.VMEM((B,), jnp.float32),
        pltpu.VMEM((B,), jnp.float32),
        pltpu.VMEM_SHARED((S_PAD,), jnp.float32),
    ],
    compiler_params=pltpu.CompilerParams(needs_layout_passes=False),
)
def _sc_counts(seg_hbm, cnt_hbm, seg_v, ones_v, zero_v, cnt_sh):
  c = lax.axis_index("c")
  s = lax.axis_index("s")
  w = c * NS + s

  _fill1d(zero_v, B, 0.0)
  _fill1d(ones_v, B, 1.0)

  def zero_body(k, carry):
    pltpu.sync_copy(zero_v, cnt_sh.at[pl.ds(s * CPS + k * B, B)])
    return carry
  lax.fori_loop(0, CPS // B, zero_body, 0)
  plsc.subcore_barrier()

  def acc_body(b, carry):
    pltpu.sync_copy(seg_hbm.at[pl.ds(w * EPW + b * B, B)], seg_v)
    pltpu.sync_copy(ones_v, cnt_sh.at[seg_v], add=True)
    return carry
  lax.fori_loop(0, NB, acc_body, 0)
  plsc.subcore_barrier()

  pltpu.sync_copy(cnt_sh.at[pl.ds(s * CPS, CPS)], cnt_hbm.at[c, pl.ds(s * CPS, CPS)])


@functools.partial(
    pl.kernel,
    out_type=jax.ShapeDtypeStruct((NC, AGG_N, D), jnp.float32),
    mesh=_mesh,
    scratch_types=[
        pltpu.VMEM((2, 3, B), jnp.int32),
        pltpu.VMEM((2, B), jnp.float32),
        pltpu.VMEM((2, B, D), jnp.float32),
        pltpu.SemaphoreType.DMA,
        pltpu.SemaphoreType.DMA,
        pltpu.SemaphoreType.DMA,
        pltpu.SemaphoreType.DMA,
        pltpu.SemaphoreType.DMA,
        pltpu.SemaphoreType.DMA,
        pltpu.VMEM_SHARED((AGG_N, D), jnp.float32),
    ],
    compiler_params=pltpu.CompilerParams(needs_layout_passes=False),
)
def _sc_scatter(xw_hbm, idx_hbm, inv_hbm, agg_hbm,
                idx_v, sc_v, rows_v, g0, g1, s0, s1, w0, w1, agg_sh):
  c = lax.axis_index("c")
  s = lax.axis_index("s")
  w = c * NS + s
  gsem = (g0, g1)
  ssem = (s0, s1)
  wsem = (w0, w1)

  _zero_rows(rows_v.at[0])
  def zero_body(k, carry):
    pltpu.sync_copy(rows_v.at[0], agg_sh.at[pl.ds(s * APS + k * B, B)])
    return carry
  lax.fori_loop(0, APS // B, zero_body, 0)

  plsc.subcore_barrier()

  def scale_rows(p):
    def one_edge(e, carry2):
      spl = plsc.load_gather(sc_v.at[p], [jnp.full((16,), e, jnp.int32)])
      for k in range(D // 16):
        rows_v[p, e, pl.ds(k * 16, 16)] = rows_v[p, e, pl.ds(k * 16, 16)] * spl
      return carry2
    lax.fori_loop(0, B, one_edge, 0, unroll=4)

  def edge_pair(i, carry):
    cps = []
    for p in range(2):
      g = 2 * i + p
      pltpu.sync_copy(idx_hbm.at[w * NB + g], idx_v.at[p])
      rcp = pltpu.async_copy(xw_hbm.at[idx_v.at[p, 0]], rows_v.at[p], gsem[p])
      scp = pltpu.async_copy(inv_hbm.at[idx_v.at[p, 1]], sc_v.at[p], ssem[p])
      cps.append((rcp, scp))
    wcps = []
    for p in range(2):
      rcp, scp = cps[p]
      scp.wait()
      rcp.wait()
      scale_rows(p)
      wcps.append(pltpu.async_copy(
          rows_v.at[p], agg_sh.at[idx_v.at[p, 2]], wsem[p], add=True))
    for p in range(2):
      wcps[p].wait()
    return carry
  lax.fori_loop(0, NB // 2, edge_pair, 0)
  plsc.subcore_barrier()

  def dump_body(k, carry):
    pltpu.sync_copy(agg_sh.at[pl.ds(s * APS + k * B, B)],
                    agg_hbm.at[c, pl.ds(s * APS + k * B, B)])
    return carry
  lax.fori_loop(0, APS // B, dump_body, 0)


def _tc_xw_body(x_ref, w_ref, out_ref):
  out_ref[0] = jnp.dot(x_ref[...], w_ref[0],
                       preferred_element_type=jnp.float32)


def _tc_xw(x, W):
  return pl.pallas_call(
      _tc_xw_body,
      grid=(R,),
      in_specs=[
          pl.BlockSpec((N, D), lambda r: (0, 0)),
          pl.BlockSpec((1, D, D), lambda r: (r, 0, 0)),
      ],
      out_specs=pl.BlockSpec((1, N, D), lambda r: (r, 0, 0)),
      out_shape=jax.ShapeDtypeStruct((R, N, D), jnp.float32),
  )(x, W)


def _tc_inv_body(cnt_ref, inv_ref):
  c = cnt_ref[0] + cnt_ref[1]
  inv_ref[...] = 1.0 / jnp.maximum(c, 1.0)


def _tc_inv(counts):
  return pl.pallas_call(
      _tc_inv_body,
      out_shape=jax.ShapeDtypeStruct((S_PAD // 128, 128), jnp.float32),
  )(counts.reshape(2, S_PAD // 128, 128)).reshape(S_PAD)


def _tc_combine_body(agg_ref, x_ref, root_ref, b_ref, out_ref, *, relu):
  acc = agg_ref[0, :N, :] + agg_ref[1, :N, :]
  acc = acc + jnp.dot(x_ref[...], root_ref[...],
                      preferred_element_type=jnp.float32) + b_ref[...]
  out_ref[...] = jnp.maximum(acc, 0.0) if relu else acc


def _tc_combine(agg, x, root, b, relu):
  return pl.pallas_call(
      functools.partial(_tc_combine_body, relu=relu),
      out_shape=jax.ShapeDtypeStruct((N, D), jnp.float32),
  )(agg, x, root, b.reshape(1, D))


def kernel(x, edge_index, edge_type, W1, root1, b1, W2, root2, b2):
  src = edge_index[0].astype(jnp.int32)
  dst = edge_index[1].astype(jnp.int32)
  et = edge_type.astype(jnp.int32)

  pad = E_PAD - E
  widx_p = jnp.concatenate([et * N + src, jnp.zeros((pad,), jnp.int32)])
  seg_p = jnp.concatenate([dst * R + et, jnp.full((pad,), SEG_DUMP, jnp.int32)])
  dst_p = jnp.concatenate([dst, jnp.full((pad,), N, jnp.int32)])
  idx_pack = jnp.stack([widx_p.reshape(-1, B), seg_p.reshape(-1, B),
                        dst_p.reshape(-1, B)], axis=1)

  counts = _sc_counts(seg_p)
  inv = _tc_inv(counts)

  xw1 = _tc_xw(x, W1).reshape(R * N, D)
  agg1 = _sc_scatter(xw1, idx_pack, inv)
  h = _tc_combine(agg1, x, root1, b1, relu=True)

  xw2 = _tc_xw(h, W2).reshape(R * N, D)
  agg2 = _sc_scatter(xw2, idx_pack, inv)
  out = _tc_combine(agg2, h, root2, b2, relu=False)
  return out


# speedup vs baseline: 2.2946x; 2.2034x over previous
import functools
import jax
import jax.numpy as jnp
from jax import lax
from jax.experimental import pallas as pl
from jax.experimental.pallas import tpu as pltpu
from jax.experimental.pallas import tpu_sc as plsc

N = 10000
E = 320000
D = 128
R = 8

NC = 2
NS = 16
NW = NC * NS

B = 128
EPW = 10240
NB = EPW // B
E_PAD = NW * EPW
CH = 8
NCHUNK = NB // CH

S_PAD = 81920
SEG_DUMP = N * R
CPS = S_PAD // NS

AGG_N = 10240
APS = AGG_N // NS

_mesh = plsc.VectorSubcoreMesh(core_axis_name="c", subcore_axis_name="s")


def _fill1d(ref, n, val):
  def body(i, carry):
    ref[pl.ds(i * 16, 16)] = jnp.full((16,), val, dtype=ref.dtype)
    return carry
  lax.fori_loop(0, n // 16, body, 0)


def _zero_rows(ref):
  def body(i, carry):
    e = i // (D // 16)
    k = i % (D // 16)
    ref[e, pl.ds(k * 16, 16)] = jnp.zeros((16,), dtype=ref.dtype)
    return carry
  lax.fori_loop(0, B * (D // 16), body, 0)


@functools.partial(
    pl.kernel,
    out_type=jax.ShapeDtypeStruct((NC, S_PAD), jnp.float32),
    mesh=_mesh,
    scratch_types=[
        pltpu.VMEM((B,), jnp.int32),
        pltpu.VMEM((B,), jnp.float32),
        pltpu.VMEM((B,), jnp.float32),
        pltpu.VMEM_SHARED((S_PAD,), jnp.float32),
    ],
    compiler_params=pltpu.CompilerParams(needs_layout_passes=False),
)
def _sc_counts(seg_hbm, cnt_hbm, seg_v, ones_v, zero_v, cnt_sh):
  c = lax.axis_index("c")
  s = lax.axis_index("s")
  w = c * NS + s

  _fill1d(zero_v, B, 0.0)
  _fill1d(ones_v, B, 1.0)

  def zero_body(k, carry):
    pltpu.sync_copy(zero_v, cnt_sh.at[pl.ds(s * CPS + k * B, B)])
    return carry
  lax.fori_loop(0, CPS // B, zero_body, 0)
  plsc.subcore_barrier()

  def acc_body(b, carry):
    pltpu.sync_copy(seg_hbm.at[pl.ds(w * EPW + b * B, B)], seg_v)
    pltpu.sync_copy(ones_v, cnt_sh.at[seg_v], add=True)
    return carry
  lax.fori_loop(0, NB, acc_body, 0)
  plsc.subcore_barrier()

  pltpu.sync_copy(cnt_sh.at[pl.ds(s * CPS, CPS)], cnt_hbm.at[c, pl.ds(s * CPS, CPS)])


@functools.partial(
    pl.kernel,
    out_type=jax.ShapeDtypeStruct((NC, AGG_N, D), jnp.float32),
    mesh=_mesh,
    scratch_types=[
        pltpu.VMEM((2, 3, B), jnp.int32),
        pltpu.VMEM((2, B), jnp.float32),
        pltpu.VMEM((2, B, D), jnp.float32),
        pltpu.SemaphoreType.DMA,
        pltpu.SemaphoreType.DMA,
        pltpu.SemaphoreType.DMA,
        pltpu.SemaphoreType.DMA,
        pltpu.SemaphoreType.DMA,
        pltpu.SemaphoreType.DMA,
        pltpu.VMEM_SHARED((AGG_N, D), jnp.float32),
    ],
    compiler_params=pltpu.CompilerParams(needs_layout_passes=False),
)
def _sc_scatter(xw_hbm, idx_hbm, inv_hbm, agg_hbm,
                idx_v, sc_v, rows_v, g0, g1, s0, s1, w0, w1, agg_sh):
  c = lax.axis_index("c")
  s = lax.axis_index("s")
  w = c * NS + s
  gsem = (g0, g1)
  ssem = (s0, s1)
  wsem = (w0, w1)

  _zero_rows(rows_v.at[0])
  def zero_body(k, carry):
    pltpu.sync_copy(rows_v.at[0], agg_sh.at[pl.ds(s * APS + k * B, B)])
    return carry
  lax.fori_loop(0, APS // B, zero_body, 0)

  plsc.subcore_barrier()

  def scale_rows(p):
    def one_edge(e, carry2):
      spl = plsc.load_gather(sc_v.at[p], [jnp.full((16,), e, jnp.int32)])
      for k in range(D // 16):
        rows_v[p, e, pl.ds(k * 16, 16)] = rows_v[p, e, pl.ds(k * 16, 16)] * spl
      return carry2
    lax.fori_loop(0, B, one_edge, 0, unroll=4)

  def edge_pair(i, carry):
    cps = []
    for p in range(2):
      g = 2 * i + p
      pltpu.sync_copy(idx_hbm.at[w * NB + g], idx_v.at[p])
      rcp = pltpu.async_copy(xw_hbm.at[idx_v.at[p, 0]], rows_v.at[p], gsem[p])
      scp = pltpu.async_copy(inv_hbm.at[idx_v.at[p, 1]], sc_v.at[p], ssem[p])
      cps.append((rcp, scp))
    wcps = []
    for p in range(2):
      rcp, scp = cps[p]
      scp.wait()
      rcp.wait()
      scale_rows(p)
      wcps.append(pltpu.async_copy(
          rows_v.at[p], agg_sh.at[idx_v.at[p, 2]], wsem[p], add=True))
    for p in range(2):
      wcps[p].wait()
    return carry
  lax.fori_loop(0, NB // 2, edge_pair, 0)
  plsc.subcore_barrier()

  def dump_body(k, carry):
    pltpu.sync_copy(agg_sh.at[pl.ds(s * APS + k * B, B)],
                    agg_hbm.at[c, pl.ds(s * APS + k * B, B)])
    return carry
  lax.fori_loop(0, APS // B, dump_body, 0)


def _tc_xw_body(x_ref, w_ref, out_ref):
  out_ref[0] = jnp.dot(x_ref[...], w_ref[0],
                       preferred_element_type=jnp.float32)


def _tc_xw(x, W):
  return pl.pallas_call(
      _tc_xw_body,
      grid=(R,),
      in_specs=[
          pl.BlockSpec((N, D), lambda r: (0, 0)),
          pl.BlockSpec((1, D, D), lambda r: (r, 0, 0)),
      ],
      out_specs=pl.BlockSpec((1, N, D), lambda r: (r, 0, 0)),
      out_shape=jax.ShapeDtypeStruct((R, N, D), jnp.float32),
  )(x, W)


def _tc_inv_body(cnt_ref, inv_ref):
  c = cnt_ref[0] + cnt_ref[1]
  inv_ref[...] = 1.0 / jnp.maximum(c, 1.0)


def _tc_inv(counts):
  return pl.pallas_call(
      _tc_inv_body,
      out_shape=jax.ShapeDtypeStruct((S_PAD // 128, 128), jnp.float32),
  )(counts.reshape(2, S_PAD // 128, 128)).reshape(S_PAD)


def _tc_combine_body(agg_ref, x_ref, root_ref, b_ref, out_ref, *, relu):
  acc = agg_ref[0, :N, :] + agg_ref[1, :N, :]
  acc = acc + jnp.dot(x_ref[...], root_ref[...],
                      preferred_element_type=jnp.float32) + b_ref[...]
  out_ref[...] = jnp.maximum(acc, 0.0) if relu else acc


def _tc_combine(agg, x, root, b, relu):
  return pl.pallas_call(
      functools.partial(_tc_combine_body, relu=relu),
      out_shape=jax.ShapeDtypeStruct((N, D), jnp.float32),
  )(agg, x, root, b.reshape(1, D))


def kernel(x, edge_index, edge_type, W1, root1, b1, W2, root2, b2):
  src = edge_index[0].astype(jnp.int32)
  dst = edge_index[1].astype(jnp.int32)
  et = edge_type.astype(jnp.int32)

  epw_real = E // NW
  padw = EPW - epw_real
  pad_ar = jnp.arange(padw, dtype=jnp.int32)

  def _pad(a, padv):
    a2 = a.reshape(NW, epw_real)
    p2 = jnp.broadcast_to(padv, (NW, padw))
    return jnp.concatenate([a2, p2], axis=1).reshape(-1)

  widx_p = _pad(et * N + src, pad_ar)
  seg_p = _pad(dst * R + et, SEG_DUMP + pad_ar)
  dst_p = _pad(dst, N + (pad_ar % (AGG_N - N)))
  idx_pack = jnp.stack([widx_p.reshape(-1, B), seg_p.reshape(-1, B),
                        dst_p.reshape(-1, B)], axis=1)

  counts = _sc_counts(seg_p)
  inv = _tc_inv(counts)

  xw1 = _tc_xw(x, W1).reshape(R * N, D)
  agg1 = _sc_scatter(xw1, idx_pack, inv)
  h = _tc_combine(agg1, x, root1, b1, relu=True)

  xw2 = _tc_xw(h, W2).reshape(R * N, D)
  agg2 = _sc_scatter(xw2, idx_pack, inv)
  out = _tc_combine(agg2, h, root2, b2, relu=False)
  return out


# speedup vs baseline: 2.4000x; 1.0459x over previous
import functools
import jax
import jax.numpy as jnp
from jax import lax
from jax.experimental import pallas as pl
from jax.experimental.pallas import tpu as pltpu
from jax.experimental.pallas import tpu_sc as plsc

N = 10000
E = 320000
D = 128
R = 8

NC = 2
NS = 16
NW = NC * NS

B = 128
EPW = 10240
NB = EPW // B
E_PAD = NW * EPW
CH = 8
NCHUNK = NB // CH

S_PAD = 81920
SEG_DUMP = N * R
CPS = S_PAD // NS

AGG_N = 10240
APS = AGG_N // NS

_mesh = plsc.VectorSubcoreMesh(core_axis_name="c", subcore_axis_name="s")


def _fill1d(ref, n, val):
  def body(i, carry):
    ref[pl.ds(i * 16, 16)] = jnp.full((16,), val, dtype=ref.dtype)
    return carry
  lax.fori_loop(0, n // 16, body, 0)


def _zero_rows(ref):
  def body(i, carry):
    e = i // (D // 16)
    k = i % (D // 16)
    ref[e, pl.ds(k * 16, 16)] = jnp.zeros((16,), dtype=ref.dtype)
    return carry
  lax.fori_loop(0, B * (D // 16), body, 0)


@functools.partial(
    pl.kernel,
    out_type=jax.ShapeDtypeStruct((NC, S_PAD), jnp.float32),
    mesh=_mesh,
    scratch_types=[
        pltpu.VMEM((B,), jnp.int32),
        pltpu.VMEM((B,), jnp.float32),
        pltpu.VMEM((B,), jnp.float32),
        pltpu.VMEM_SHARED((S_PAD,), jnp.float32),
    ],
    compiler_params=pltpu.CompilerParams(needs_layout_passes=False),
)
def _sc_counts(seg_hbm, cnt_hbm, seg_v, ones_v, zero_v, cnt_sh):
  c = lax.axis_index("c")
  s = lax.axis_index("s")
  w = c * NS + s

  _fill1d(zero_v, B, 0.0)
  _fill1d(ones_v, B, 1.0)

  def zero_body(k, carry):
    pltpu.sync_copy(zero_v, cnt_sh.at[pl.ds(s * CPS + k * B, B)])
    return carry
  lax.fori_loop(0, CPS // B, zero_body, 0)
  plsc.subcore_barrier()

  def acc_body(b, carry):
    pltpu.sync_copy(seg_hbm.at[pl.ds(w * EPW + b * B, B)], seg_v)
    pltpu.sync_copy(ones_v, cnt_sh.at[seg_v], add=True)
    return carry
  lax.fori_loop(0, NB, acc_body, 0)
  plsc.subcore_barrier()

  pltpu.sync_copy(cnt_sh.at[pl.ds(s * CPS, CPS)], cnt_hbm.at[c, pl.ds(s * CPS, CPS)])


@functools.partial(
    pl.kernel,
    out_type=jax.ShapeDtypeStruct((NC, AGG_N, D), jnp.float32),
    mesh=_mesh,
    scratch_types=[
        pltpu.VMEM((4, 3, B), jnp.int32),
        pltpu.VMEM((2, B), jnp.float32),
        pltpu.VMEM((2, B, D), jnp.float32),
        pltpu.SemaphoreType.DMA,
        pltpu.SemaphoreType.DMA,
        pltpu.SemaphoreType.DMA,
        pltpu.SemaphoreType.DMA,
        pltpu.SemaphoreType.DMA,
        pltpu.SemaphoreType.DMA,
        pltpu.SemaphoreType.DMA,
        pltpu.SemaphoreType.DMA,
        pltpu.SemaphoreType.DMA,
        pltpu.SemaphoreType.DMA,
        pltpu.VMEM_SHARED((AGG_N, D), jnp.float32),
    ],
    compiler_params=pltpu.CompilerParams(needs_layout_passes=False),
)
def _sc_scatter(xw_hbm, idx_hbm, inv_hbm, agg_hbm,
                idx_v, sc_v, rows_v, g0, g1, s0, s1, w0, w1,
                i0, i1, i2, i3, agg_sh):
  c = lax.axis_index("c")
  s = lax.axis_index("s")
  w = c * NS + s
  gsem = (g0, g1)
  ssem = (s0, s1)
  wsem = (w0, w1)
  isem = (i0, i1, i2, i3)

  _zero_rows(rows_v.at[0])
  def zero_body(k, carry):
    pltpu.sync_copy(rows_v.at[0], agg_sh.at[pl.ds(s * APS + k * B, B)])
    return carry
  lax.fori_loop(0, APS // B, zero_body, 0)

  plsc.subcore_barrier()

  def scale_rows(p):
    def one_edge(e, carry2):
      spl = plsc.load_gather(sc_v.at[p], [jnp.full((16,), e, jnp.int32)])
      for k in range(D // 16):
        rows_v[p, e, pl.ds(k * 16, 16)] = rows_v[p, e, pl.ds(k * 16, 16)] * spl
      return carry2
    lax.fori_loop(0, B, one_edge, 0, unroll=4)

  def quad_body(i, carry):
    icps = []
    for q in range(4):
      g = 4 * i + q
      icps.append(pltpu.async_copy(idx_hbm.at[w * NB + g], idx_v.at[q],
                                   isem[q]))

    def fire_gathers(q):
      p = q % 2
      icps[q].wait()
      rcp = pltpu.async_copy(xw_hbm.at[idx_v.at[q, 0]], rows_v.at[p], gsem[p])
      scp = pltpu.async_copy(inv_hbm.at[idx_v.at[q, 1]], sc_v.at[p], ssem[p])
      return rcp, scp

    def process(q, cps):
      p = q % 2
      rcp, scp = cps
      scp.wait()
      rcp.wait()
      scale_rows(p)
      return pltpu.async_copy(rows_v.at[p], agg_sh.at[idx_v.at[q, 2]],
                              wsem[p], add=True)

    cps0 = fire_gathers(0)
    cps1 = fire_gathers(1)
    wa0 = process(0, cps0)
    wa1 = process(1, cps1)
    wa0.wait()
    cps2 = fire_gathers(2)
    wa1.wait()
    cps3 = fire_gathers(3)
    wb0 = process(2, cps2)
    wb1 = process(3, cps3)
    wb0.wait()
    wb1.wait()
    return carry
  lax.fori_loop(0, NB // 4, quad_body, 0)
  plsc.subcore_barrier()

  def dump_body(k, carry):
    pltpu.sync_copy(agg_sh.at[pl.ds(s * APS + k * B, B)],
                    agg_hbm.at[c, pl.ds(s * APS + k * B, B)])
    return carry
  lax.fori_loop(0, APS // B, dump_body, 0)


def _tc_xw_body(x_ref, w_ref, out_ref):
  out_ref[0] = jnp.dot(x_ref[...], w_ref[0],
                       preferred_element_type=jnp.float32)


def _tc_xw(x, W):
  return pl.pallas_call(
      _tc_xw_body,
      grid=(R,),
      in_specs=[
          pl.BlockSpec((N, D), lambda r: (0, 0)),
          pl.BlockSpec((1, D, D), lambda r: (r, 0, 0)),
      ],
      out_specs=pl.BlockSpec((1, N, D), lambda r: (r, 0, 0)),
      out_shape=jax.ShapeDtypeStruct((R, N, D), jnp.float32),
  )(x, W)


def _tc_inv_body(cnt_ref, inv_ref):
  c = cnt_ref[0] + cnt_ref[1]
  inv_ref[...] = 1.0 / jnp.maximum(c, 1.0)


def _tc_inv(counts):
  return pl.pallas_call(
      _tc_inv_body,
      out_shape=jax.ShapeDtypeStruct((S_PAD // 128, 128), jnp.float32),
  )(counts.reshape(2, S_PAD // 128, 128)).reshape(S_PAD)


def _tc_combine_body(agg_ref, x_ref, root_ref, b_ref, out_ref, *, relu):
  acc = agg_ref[0, :N, :] + agg_ref[1, :N, :]
  acc = acc + jnp.dot(x_ref[...], root_ref[...],
                      preferred_element_type=jnp.float32) + b_ref[...]
  out_ref[...] = jnp.maximum(acc, 0.0) if relu else acc


def _tc_combine(agg, x, root, b, relu):
  return pl.pallas_call(
      functools.partial(_tc_combine_body, relu=relu),
      out_shape=jax.ShapeDtypeStruct((N, D), jnp.float32),
  )(agg, x, root, b.reshape(1, D))


def kernel(x, edge_index, edge_type, W1, root1, b1, W2, root2, b2):
  src = edge_index[0].astype(jnp.int32)
  dst = edge_index[1].astype(jnp.int32)
  et = edge_type.astype(jnp.int32)

  epw_real = E // NW
  padw = EPW - epw_real
  pad_ar = jnp.arange(padw, dtype=jnp.int32)

  def _pad(a, padv):
    a2 = a.reshape(NW, epw_real)
    p2 = jnp.broadcast_to(padv, (NW, padw))
    return jnp.concatenate([a2, p2], axis=1).reshape(-1)

  widx_p = _pad(et * N + src, pad_ar)
  seg_p = _pad(dst * R + et, SEG_DUMP + pad_ar)
  dst_p = _pad(dst, N + (pad_ar % (AGG_N - N)))
  idx_pack = jnp.stack([widx_p.reshape(-1, B), seg_p.reshape(-1, B),
                        dst_p.reshape(-1, B)], axis=1)

  counts = _sc_counts(seg_p)
  inv = _tc_inv(counts)

  xw1 = _tc_xw(x, W1).reshape(R * N, D)
  agg1 = _sc_scatter(xw1, idx_pack, inv)
  h = _tc_combine(agg1, x, root1, b1, relu=True)

  xw2 = _tc_xw(h, W2).reshape(R * N, D)
  agg2 = _sc_scatter(xw2, idx_pack, inv)
  out = _tc_combine(agg2, h, root2, b2, relu=False)
  return out


# speedup vs baseline: 2.7560x; 1.1483x over previous
import functools
import jax
import jax.numpy as jnp
from jax import lax
from jax.experimental import pallas as pl
from jax.experimental.pallas import tpu as pltpu
from jax.experimental.pallas import tpu_sc as plsc

N = 10000
E = 320000
D = 128
R = 8

NC = 2
NS = 16
NW = NC * NS

B = 112
EPW = 10080
NB = EPW // B
E_PAD = NW * EPW

S_PAD = 81920
SEG_DUMP = N * R
CPS = S_PAD // NS

AGG_N = 10240
APS = AGG_N // NS

_mesh = plsc.VectorSubcoreMesh(core_axis_name="c", subcore_axis_name="s")


def _fill1d(ref, n, val):
  def body(i, carry):
    ref[pl.ds(i * 16, 16)] = jnp.full((16,), val, dtype=ref.dtype)
    return carry
  lax.fori_loop(0, n // 16, body, 0)


def _zero_rows(ref):
  def body(i, carry):
    e = i // (D // 16)
    k = i % (D // 16)
    ref[e, pl.ds(k * 16, 16)] = jnp.zeros((16,), dtype=ref.dtype)
    return carry
  lax.fori_loop(0, ref.shape[0] * (D // 16), body, 0)


@functools.partial(
    pl.kernel,
    out_type=jax.ShapeDtypeStruct((NC, S_PAD), jnp.float32),
    mesh=_mesh,
    scratch_types=[
        pltpu.VMEM((B,), jnp.int32),
        pltpu.VMEM((B,), jnp.float32),
        pltpu.VMEM((128,), jnp.float32),
        pltpu.VMEM_SHARED((S_PAD,), jnp.float32),
    ],
    compiler_params=pltpu.CompilerParams(needs_layout_passes=False),
)
def _sc_counts(seg_hbm, cnt_hbm, seg_v, ones_v, zero_v, cnt_sh):
  c = lax.axis_index("c")
  s = lax.axis_index("s")
  w = c * NS + s

  _fill1d(zero_v, 128, 0.0)
  _fill1d(ones_v, B, 1.0)

  def zero_body(k, carry):
    pltpu.sync_copy(zero_v, cnt_sh.at[pl.ds(s * CPS + k * 128, 128)])
    return carry
  lax.fori_loop(0, CPS // 128, zero_body, 0)
  plsc.subcore_barrier()

  def acc_body(b, carry):
    pltpu.sync_copy(seg_hbm.at[pl.ds(w * EPW + b * B, B)], seg_v)
    pltpu.sync_copy(ones_v, cnt_sh.at[seg_v], add=True)
    return carry
  lax.fori_loop(0, NB, acc_body, 0)
  plsc.subcore_barrier()

  pltpu.sync_copy(cnt_sh.at[pl.ds(s * CPS, CPS)], cnt_hbm.at[c, pl.ds(s * CPS, CPS)])


@functools.partial(
    pl.kernel,
    out_type=jax.ShapeDtypeStruct((NC, AGG_N, D), jnp.float32),
    mesh=_mesh,
    scratch_types=[
        pltpu.VMEM((6, 3, B), jnp.int32),
        pltpu.VMEM((3, B), jnp.float32),
        pltpu.VMEM((3, B, D), jnp.float32),
        [pltpu.SemaphoreType.DMA] * 3,
        [pltpu.SemaphoreType.DMA] * 3,
        [pltpu.SemaphoreType.DMA] * 3,
        [pltpu.SemaphoreType.DMA] * 6,
        pltpu.VMEM_SHARED((AGG_N, D), jnp.float32),
    ],
    compiler_params=pltpu.CompilerParams(needs_layout_passes=False),
)
def _sc_scatter(xw_hbm, idx_hbm, inv_hbm, agg_hbm,
                idx_v, sc_v, rows_v, gsem, ssem, wsem, isem, agg_sh):
  c = lax.axis_index("c")
  s = lax.axis_index("s")
  w = c * NS + s

  _zero_rows(rows_v.at[0])
  for k in range(APS // B):
    pltpu.sync_copy(rows_v.at[0], agg_sh.at[pl.ds(s * APS + k * B, B)])
  rem = APS - (APS // B) * B
  if rem:
    pltpu.sync_copy(rows_v.at[0, pl.ds(0, rem)],
                    agg_sh.at[pl.ds(s * APS + (APS // B) * B, rem)])

  plsc.subcore_barrier()

  def scale_rows(p):
    def one_edge(e, carry2):
      spl = plsc.load_gather(sc_v.at[p], [jnp.full((16,), e, jnp.int32)])
      for k in range(D // 16):
        rows_v[p, e, pl.ds(k * 16, 16)] = rows_v[p, e, pl.ds(k * 16, 16)] * spl
      return carry2
    lax.fori_loop(0, B, one_edge, 0, unroll=4)

  def ring_body(i, carry):
    icps = []
    for q in range(6):
      g = 6 * i + q
      icps.append(pltpu.async_copy(idx_hbm.at[w * NB + g], idx_v.at[q],
                                   isem[q]))

    def fire_gathers(q):
      p = q % 3
      icps[q].wait()
      rcp = pltpu.async_copy(xw_hbm.at[idx_v.at[q, 0]], rows_v.at[p],
                             gsem[p])
      scp = pltpu.async_copy(inv_hbm.at[idx_v.at[q, 1]], sc_v.at[p],
                             ssem[p])
      return rcp, scp

    def process(q, cps):
      p = q % 3
      rcp, scp = cps
      scp.wait()
      rcp.wait()
      scale_rows(p)
      return pltpu.async_copy(rows_v.at[p], agg_sh.at[idx_v.at[q, 2]],
                              wsem[p], add=True)

    c0 = fire_gathers(0)
    c1 = fire_gathers(1)
    c2 = fire_gathers(2)
    w0 = process(0, c0)
    w1 = process(1, c1)
    w0.wait()
    c3 = fire_gathers(3)
    w2 = process(2, c2)
    w1.wait()
    c4 = fire_gathers(4)
    w3 = process(3, c3)
    w2.wait()
    c5 = fire_gathers(5)
    w4 = process(4, c4)
    w3.wait()
    w5 = process(5, c5)
    w4.wait()
    w5.wait()
    return carry
  lax.fori_loop(0, NB // 6, ring_body, 0)
  plsc.subcore_barrier()

  def dump_body(k, carry):
    pltpu.sync_copy(agg_sh.at[pl.ds(s * APS + k * B, B)],
                    agg_hbm.at[c, pl.ds(s * APS + k * B, B)])
    return carry
  lax.fori_loop(0, APS // B, dump_body, 0)


def _tc_xw_body(x_ref, w_ref, out_ref):
  out_ref[0] = jnp.dot(x_ref[...], w_ref[0],
                       preferred_element_type=jnp.float32)


def _tc_xw(x, W):
  return pl.pallas_call(
      _tc_xw_body,
      grid=(R,),
      in_specs=[
          pl.BlockSpec((N, D), lambda r: (0, 0)),
          pl.BlockSpec((1, D, D), lambda r: (r, 0, 0)),
      ],
      out_specs=pl.BlockSpec((1, N, D), lambda r: (r, 0, 0)),
      out_shape=jax.ShapeDtypeStruct((R, N, D), jnp.float32),
  )(x, W)


def _tc_inv_body(cnt_ref, inv_ref):
  c = cnt_ref[0] + cnt_ref[1]
  inv_ref[...] = 1.0 / jnp.maximum(c, 1.0)


def _tc_inv(counts):
  return pl.pallas_call(
      _tc_inv_body,
      out_shape=jax.ShapeDtypeStruct((S_PAD // 128, 128), jnp.float32),
  )(counts.reshape(2, S_PAD // 128, 128)).reshape(S_PAD)


def _tc_combine_body(agg_ref, x_ref, root_ref, b_ref, out_ref, *, relu):
  acc = agg_ref[0, :N, :] + agg_ref[1, :N, :]
  acc = acc + jnp.dot(x_ref[...], root_ref[...],
                      preferred_element_type=jnp.float32) + b_ref[...]
  out_ref[...] = jnp.maximum(acc, 0.0) if relu else acc


def _tc_combine(agg, x, root, b, relu):
  return pl.pallas_call(
      functools.partial(_tc_combine_body, relu=relu),
      out_shape=jax.ShapeDtypeStruct((N, D), jnp.float32),
  )(agg, x, root, b.reshape(1, D))


def kernel(x, edge_index, edge_type, W1, root1, b1, W2, root2, b2):
  src = edge_index[0].astype(jnp.int32)
  dst = edge_index[1].astype(jnp.int32)
  et = edge_type.astype(jnp.int32)

  epw_real = E // NW
  padw = EPW - epw_real
  pad_ar = jnp.arange(padw, dtype=jnp.int32)

  def _pad(a, padv):
    a2 = a.reshape(NW, epw_real)
    p2 = jnp.broadcast_to(padv, (NW, padw))
    return jnp.concatenate([a2, p2], axis=1).reshape(-1)

  widx_p = _pad(et * N + src, pad_ar)
  seg_p = _pad(dst * R + et, SEG_DUMP + pad_ar)
  dst_p = _pad(dst, N + (pad_ar % (AGG_N - N)))
  idx_pack = jnp.stack([widx_p.reshape(-1, B), seg_p.reshape(-1, B),
                        dst_p.reshape(-1, B)], axis=1)

  counts = _sc_counts(seg_p)
  inv = _tc_inv(counts)

  xw1 = _tc_xw(x, W1).reshape(R * N, D)
  agg1 = _sc_scatter(xw1, idx_pack, inv)
  h = _tc_combine(agg1, x, root1, b1, relu=True)

  xw2 = _tc_xw(h, W2).reshape(R * N, D)
  agg2 = _sc_scatter(xw2, idx_pack, inv)
  out = _tc_combine(agg2, h, root2, b2, relu=False)
  return out
